# baseline (speedup 1.0000x reference)
import functools

import jax
import jax.numpy as jnp
from jax import lax
from jax.experimental import pallas as pl
from jax.experimental.pallas import tpu as pltpu

_LANES = 512
_CHUNK = 512


def _block_body(x_ref, t_ref, o_ref, *, rows_per_block, valid_last):
    def block_sum(mask_rem):
        acc = jnp.zeros((8, 128), jnp.float32)
        for c in range(rows_per_block // _CHUNK):
            r0 = c * _CHUNK
            x = x_ref[r0:r0 + _CHUNK, :]
            t = t_ref[r0:r0 + _CHUNK, :]
            bce = jnp.maximum(x, 0.0) - x * t + jnp.log(1.0 + jnp.exp(-jnp.abs(x)))
            if mask_rem is not None:
                row = lax.broadcasted_iota(jnp.int32, (_CHUNK, _LANES), 0)
                col = lax.broadcasted_iota(jnp.int32, (_CHUNK, _LANES), 1)
                flat = (r0 + row) * _LANES + col
                bce = jnp.where(flat < mask_rem, bce, 0.0)
            narrow = bce[:, 0:128]
            for j in range(1, _LANES // 128):
                narrow = narrow + bce[:, j * 128:(j + 1) * 128]
            folded = narrow[0:8, :]
            for r in range(1, _CHUNK // 8):
                folded = folded + narrow[r * 8:(r + 1) * 8, :]
            acc = acc + folded
        return acc[None, :, :]

    if valid_last is None:
        o_ref[...] = block_sum(None)
    else:
        last = pl.num_programs(0) - 1

        @pl.when(pl.program_id(0) != last)
        def _():
            o_ref[...] = block_sum(None)

        @pl.when(pl.program_id(0) == last)
        def _():
            o_ref[...] = block_sum(valid_last)


def _bce_mean(inputs: jax.Array, targets: jax.Array) -> jax.Array:
    total = int(inputs.size)
    rows = pl.cdiv(total, _LANES)

    max_rpb = 2048
    rpb = min(max_rpb, max(_CHUNK, (rows // 2 // _CHUNK) * _CHUNK))
    num_blocks = pl.cdiv(rows, rpb)
    padded_rows = num_blocks * rpb
    rem = total - (num_blocks - 1) * rpb * _LANES
    valid_last = None if rem == rpb * _LANES else rem

    def _as2d(a):
        flat = jnp.reshape(a, (-1,))
        pad = padded_rows * _LANES - total
        if pad:
            flat = jnp.pad(flat, (0, pad))
        return jnp.reshape(flat, (padded_rows, _LANES))

    x2 = _as2d(inputs)
    t2 = _as2d(targets)

    body = functools.partial(
        _block_body, rows_per_block=rpb, valid_last=valid_last)

    partials = pl.pallas_call(
        body,
        out_shape=jax.ShapeDtypeStruct((num_blocks, 8, 128), jnp.float32),
        grid=(num_blocks,),
        in_specs=[
            pl.BlockSpec((rpb, _LANES), lambda i: (i, 0)),
            pl.BlockSpec((rpb, _LANES), lambda i: (i, 0)),
        ],
        out_specs=pl.BlockSpec((1, 8, 128), lambda i: (i, 0, 0)),
        compiler_params=pltpu.CompilerParams(
            dimension_semantics=("parallel",),
            vmem_limit_bytes=60 << 20,
        ),
        cost_estimate=pl.CostEstimate(
            flops=7 * total,
            transcendentals=2 * total,
            bytes_accessed=int(2 * total * 4 + num_blocks * 8 * 128 * 4),
        ),
    )(x2, t2)

    return jnp.sum(partials) / jnp.float32(total)


def kernel(inputs, targets):
    return _bce_mean(inputs, targets)


# speedup vs baseline: 3.5212x; 3.5212x over previous
import functools

import jax
import jax.numpy as jnp
from jax import lax
from jax.experimental import pallas as pl
from jax.experimental.pallas import tpu as pltpu

_CHUNK = 1024


def _block_body(x_ref, t_ref, o_ref, *, rows_per_block, lanes, valid_last):
    chunk = min(_CHUNK, rows_per_block)

    def block_sum(mask_rem):
        acc = jnp.zeros((8, 128), jnp.float32)
        for c in range(rows_per_block // chunk):
            r0 = c * chunk
            x = x_ref[r0:r0 + chunk, :]
            t = t_ref[r0:r0 + chunk, :]
            bce = jnp.maximum(x, 0.0) - x * t + jnp.log(1.0 + jnp.exp(-jnp.abs(x)))
            if mask_rem is not None:
                row = lax.broadcasted_iota(jnp.int32, (chunk, lanes), 0)
                col = lax.broadcasted_iota(jnp.int32, (chunk, lanes), 1)
                flat = (r0 + row) * lanes + col
                bce = jnp.where(flat < mask_rem, bce, 0.0)
            narrow = bce[:, 0:128]
            for j in range(1, lanes // 128):
                narrow = narrow + bce[:, j * 128:(j + 1) * 128]
            folded = narrow[0:8, :]
            for r in range(1, chunk // 8):
                folded = folded + narrow[r * 8:(r + 1) * 8, :]
            acc = acc + folded
        return acc[None, :, :]

    if valid_last is None:
        o_ref[...] = block_sum(None)
    else:
        last = pl.num_programs(0) - 1

        @pl.when(pl.program_id(0) != last)
        def _():
            o_ref[...] = block_sum(None)

        @pl.when(pl.program_id(0) == last)
        def _():
            o_ref[...] = block_sum(valid_last)


def _bce_mean(inputs: jax.Array, targets: jax.Array) -> jax.Array:
    total = int(inputs.size)

    if inputs.ndim >= 2 and inputs.shape[-1] % 128 == 0 and (
            total // inputs.shape[-1]) % 8 == 0:
        lanes = inputs.shape[-1]
    else:
        lanes = 128
    rows = pl.cdiv(total, lanes)

    target_rows = max(8, (4 << 20) // (lanes * 4))
    num_blocks = max(2, pl.cdiv(rows, target_rows))
    rpb = pl.cdiv(rows, num_blocks)
    rpb = (rpb + 7) // 8 * 8
    num_blocks = pl.cdiv(rows, rpb)
    padded_rows = num_blocks * rpb
    rem = total - (num_blocks - 1) * rpb * lanes
    valid_last = None if rem == rpb * lanes else rem

    def _as2d(a):
        flat = jnp.reshape(a, (-1,))
        pad = padded_rows * lanes - total
        if pad:
            flat = jnp.pad(flat, (0, pad))
        return jnp.reshape(flat, (padded_rows, lanes))

    x2 = _as2d(inputs)
    t2 = _as2d(targets)

    body = functools.partial(
        _block_body, rows_per_block=rpb, lanes=lanes, valid_last=valid_last)

    partials = pl.pallas_call(
        body,
        out_shape=jax.ShapeDtypeStruct((num_blocks, 8, 128), jnp.float32),
        grid=(num_blocks,),
        in_specs=[
            pl.BlockSpec((rpb, lanes), lambda i: (i, 0)),
            pl.BlockSpec((rpb, lanes), lambda i: (i, 0)),
        ],
        out_specs=pl.BlockSpec((1, 8, 128), lambda i: (i, 0, 0)),
        compiler_params=pltpu.CompilerParams(
            dimension_semantics=("parallel",),
            vmem_limit_bytes=60 << 20,
        ),
        cost_estimate=pl.CostEstimate(
            flops=7 * total,
            transcendentals=2 * total,
            bytes_accessed=int(2 * total * 4 + num_blocks * 8 * 128 * 4),
        ),
    )(x2, t2)

    return jnp.sum(partials) / jnp.float32(total)


def kernel(inputs, targets):
    return _bce_mean(inputs, targets)
